# baseline (speedup 1.0000x reference)
import jax
import jax.numpy as jnp
from jax.experimental import pallas as pl
from jax.experimental.pallas import tpu as pltpu

_CH = 64


def _ce_uni_big(y, j, desc):
    n, r, dh = y.shape
    y3 = y.reshape(n * (r // (2 * j)), 2 * j, dh)
    a = y3[:, :j]
    b = y3[:, j:]
    lo = jnp.minimum(a, b)
    hi = jnp.maximum(a, b)
    if desc:
        lo, hi = hi, lo
    return jnp.concatenate([lo, hi], axis=1).reshape(n, r, dh)


def _ce_uni_small(y, j, desc):
    n, r, dh = y.shape
    z = y.reshape(n * (r // 8), 8, dh)
    i8 = jax.lax.broadcasted_iota(jnp.int32, (1, 8, 1), 1)
    t1 = (i8 & j) == 0
    rm = pltpu.roll(z, 8 - j, 1)
    rp = rm if j == 4 else pltpu.roll(z, j, 1)
    if desc:
        out = jnp.where(t1, jnp.maximum(z, rm), jnp.minimum(z, rp))
    else:
        out = jnp.where(t1, jnp.minimum(z, rm), jnp.maximum(z, rp))
    return out.reshape(n, r, dh)


def _ce_uni(y, j, desc):
    return _ce_uni_big(y, j, desc) if j >= 8 else _ce_uni_small(y, j, desc)


def _ce(x, j, k, flip):
    ch, dh = x.shape
    if k >= ch:
        return _ce_uni(x[None], j, flip)[0]
    if k >= 8:
        x4 = x.reshape(ch // (2 * k), 2, k, dh)
        asc = _ce_uni(x4[:, 0], j, flip)
        dsc = _ce_uni(x4[:, 1], j, not flip)
        return jnp.stack([asc, dsc], axis=1).reshape(ch, dh)
    z = x.reshape(ch // 8, 8, dh)
    i8 = jax.lax.broadcasted_iota(jnp.int32, (1, 8, 1), 1)
    t1 = (i8 & j) == 0
    keep_min = t1 ^ ((i8 & k) != 0)
    if flip:
        keep_min = ~keep_min
    partner = jnp.where(t1, pltpu.roll(z, 8 - j, 1), pltpu.roll(z, j, 1))
    lo = jnp.minimum(z, partner)
    hi = jnp.maximum(z, partner)
    return jnp.where(keep_min, lo, hi).reshape(ch, dh)


def _bitonic_kernel(v_ref, mask_ref, out_ref):
    s = v_ref.shape[2]
    dh = v_ref.shape[3]
    log_s = s.bit_length() - 1
    ch = _CH
    log_ch = ch.bit_length() - 1
    nch = s // ch

    sign = jnp.where(mask_ref[0:1, :] != 0, jnp.float32(-1.0), jnp.float32(1.0))

    def chunk_sort(x, desc):
        for kk in range(1, log_ch + 1):
            k = 1 << kk
            flip = desc if k >= ch else False
            for jj in range(kk - 1, -1, -1):
                x = _ce(x, 1 << jj, k, flip)
        return x

    def chunk_merge(x, desc):
        for jj in range(log_ch - 1, -1, -1):
            x = _ce(x, 1 << jj, s, desc)
        return x

    def pass_sort2(p, desc):
        base = p * (2 * ch)
        x0 = v_ref[0, 0, pl.ds(base, ch), :] * sign
        x1 = v_ref[0, 0, pl.ds(base + ch, ch), :] * sign
        x0 = chunk_sort(x0, False)
        x1 = chunk_sort(x1, True)
        lo = jnp.minimum(x0, x1)
        hi = jnp.maximum(x0, x1)
        if desc:
            lo, hi = hi, lo
        out_ref[0, 0, pl.ds(base, ch), :] = chunk_merge(lo, desc)
        out_ref[0, 0, pl.ds(base + ch, ch), :] = chunk_merge(hi, desc)

    def sort2_asc(t, carry):
        pass_sort2(4 * t, False)
        pass_sort2(4 * t + 2, False)
        return carry

    def sort2_desc(t, carry):
        pass_sort2(4 * t + 1, True)
        pass_sort2(4 * t + 3, True)
        return carry

    jax.lax.fori_loop(0, nch // 8, sort2_asc, 0)
    jax.lax.fori_loop(0, nch // 8, sort2_desc, 0)

    def pass_merge(g, span, desc, unsign):
        base = g * span * ch
        xs = [out_ref[0, 0, pl.ds(base + i * ch, ch), :] for i in range(span)]
        dist = span // 2
        while dist >= 1:
            for blk in range(0, span, 2 * dist):
                for i in range(blk, blk + dist):
                    lo = jnp.minimum(xs[i], xs[i + dist])
                    hi = jnp.maximum(xs[i], xs[i + dist])
                    xs[i], xs[i + dist] = (hi, lo) if desc else (lo, hi)
            dist //= 2
        for i in range(span):
            y = chunk_merge(xs[i], desc)
            if unsign:
                y = y * sign
            out_ref[0, 0, pl.ds(base + i * ch, ch), :] = y

    span = 4
    while span <= nch:
        last = span == nch
        ngroups = nch // span

        def m_asc(t, carry, span=span, last=last):
            pass_merge((1 if last else 2) * t, span, False, last)
            return carry

        def m_desc(t, carry, span=span):
            pass_merge(2 * t + 1, span, True, False)
            return carry

        jax.lax.fori_loop(0, ngroups if last else ngroups // 2, m_asc, 0)
        if not last:
            jax.lax.fori_loop(0, ngroups // 2, m_desc, 0)
        span *= 2


def _sort_modulated(v, mask):
    b, h, s, dh = v.shape
    return pl.pallas_call(
        _bitonic_kernel,
        grid=(b, h),
        in_specs=[
            pl.BlockSpec((1, 1, s, dh), lambda i, j: (i, j, 0, 0)),
            pl.BlockSpec((8, dh), lambda i, j: (0, 0)),
        ],
        out_specs=pl.BlockSpec((1, 1, s, dh), lambda i, j: (i, j, 0, 0)),
        out_shape=jax.ShapeDtypeStruct(v.shape, v.dtype),
    )(v, mask)


@jax.jit
def kernel(q, k, v, col_descend):
    del q, k
    dh = v.shape[-1]
    cols = jnp.asarray(col_descend).reshape(-1).astype(jnp.int32)
    mask = jnp.zeros((8, dh), jnp.int32).at[0, cols].set(1)
    out = _sort_modulated(v, mask)
    return (out, out)


# speedup vs baseline: 1.0191x; 1.0191x over previous
import jax
import jax.numpy as jnp
from jax.experimental import pallas as pl
from jax.experimental.pallas import tpu as pltpu

_CH = 128


def _ce_uni_big(y, j, desc):
    n, r, dh = y.shape
    y3 = y.reshape(n * (r // (2 * j)), 2 * j, dh)
    a = y3[:, :j]
    b = y3[:, j:]
    lo = jnp.minimum(a, b)
    hi = jnp.maximum(a, b)
    if desc:
        lo, hi = hi, lo
    return jnp.concatenate([lo, hi], axis=1).reshape(n, r, dh)


def _ce_uni_small(y, j, desc):
    n, r, dh = y.shape
    z = y.reshape(n * (r // 8), 8, dh)
    i8 = jax.lax.broadcasted_iota(jnp.int32, (1, 8, 1), 1)
    t1 = (i8 & j) == 0
    rm = pltpu.roll(z, 8 - j, 1)
    rp = rm if j == 4 else pltpu.roll(z, j, 1)
    if desc:
        out = jnp.where(t1, jnp.maximum(z, rm), jnp.minimum(z, rp))
    else:
        out = jnp.where(t1, jnp.minimum(z, rm), jnp.maximum(z, rp))
    return out.reshape(n, r, dh)


def _ce_uni(y, j, desc):
    return _ce_uni_big(y, j, desc) if j >= 8 else _ce_uni_small(y, j, desc)


def _ce(x, j, k, flip):
    ch, dh = x.shape
    if k >= ch:
        return _ce_uni(x[None], j, flip)[0]
    if k >= 8:
        x4 = x.reshape(ch // (2 * k), 2, k, dh)
        asc = _ce_uni(x4[:, 0], j, flip)
        dsc = _ce_uni(x4[:, 1], j, not flip)
        return jnp.stack([asc, dsc], axis=1).reshape(ch, dh)
    z = x.reshape(ch // 8, 8, dh)
    i8 = jax.lax.broadcasted_iota(jnp.int32, (1, 8, 1), 1)
    t1 = (i8 & j) == 0
    keep_min = t1 ^ ((i8 & k) != 0)
    if flip:
        keep_min = ~keep_min
    partner = jnp.where(t1, pltpu.roll(z, 8 - j, 1), pltpu.roll(z, j, 1))
    lo = jnp.minimum(z, partner)
    hi = jnp.maximum(z, partner)
    return jnp.where(keep_min, lo, hi).reshape(ch, dh)


def _bitonic_kernel(v_ref, mask_ref, out_ref):
    s = v_ref.shape[2]
    dh = v_ref.shape[3]
    log_s = s.bit_length() - 1
    ch = _CH
    log_ch = ch.bit_length() - 1
    nch = s // ch

    sign = jnp.where(mask_ref[0:1, :] != 0, jnp.float32(-1.0), jnp.float32(1.0))

    def chunk_sort(x, desc):
        for kk in range(1, log_ch + 1):
            k = 1 << kk
            flip = desc if k >= ch else False
            for jj in range(kk - 1, -1, -1):
                x = _ce(x, 1 << jj, k, flip)
        return x

    def chunk_merge(x, desc):
        for jj in range(log_ch - 1, -1, -1):
            x = _ce(x, 1 << jj, s, desc)
        return x

    def pass_sort2(p, desc):
        base = p * (2 * ch)
        x0 = v_ref[0, 0, pl.ds(base, ch), :] * sign
        x1 = v_ref[0, 0, pl.ds(base + ch, ch), :] * sign
        x0 = chunk_sort(x0, False)
        x1 = chunk_sort(x1, True)
        lo = jnp.minimum(x0, x1)
        hi = jnp.maximum(x0, x1)
        if desc:
            lo, hi = hi, lo
        out_ref[0, 0, pl.ds(base, ch), :] = chunk_merge(lo, desc)
        out_ref[0, 0, pl.ds(base + ch, ch), :] = chunk_merge(hi, desc)

    def sort2_asc(t, carry):
        pass_sort2(4 * t, False)
        pass_sort2(4 * t + 2, False)
        return carry

    def sort2_desc(t, carry):
        pass_sort2(4 * t + 1, True)
        pass_sort2(4 * t + 3, True)
        return carry

    jax.lax.fori_loop(0, nch // 8, sort2_asc, 0)
    jax.lax.fori_loop(0, nch // 8, sort2_desc, 0)

    def pass_merge(g, span, desc, unsign):
        base = g * span * ch
        xs = [out_ref[0, 0, pl.ds(base + i * ch, ch), :] for i in range(span)]
        dist = span // 2
        while dist >= 1:
            for blk in range(0, span, 2 * dist):
                for i in range(blk, blk + dist):
                    lo = jnp.minimum(xs[i], xs[i + dist])
                    hi = jnp.maximum(xs[i], xs[i + dist])
                    xs[i], xs[i + dist] = (hi, lo) if desc else (lo, hi)
            dist //= 2
        for i in range(span):
            y = chunk_merge(xs[i], desc)
            if unsign:
                y = y * sign
            out_ref[0, 0, pl.ds(base + i * ch, ch), :] = y

    span = 4
    while span <= nch:
        last = span == nch
        ngroups = nch // span

        def m_asc(t, carry, span=span, last=last):
            pass_merge((1 if last else 2) * t, span, False, last)
            return carry

        def m_desc(t, carry, span=span):
            pass_merge(2 * t + 1, span, True, False)
            return carry

        jax.lax.fori_loop(0, ngroups if last else ngroups // 2, m_asc, 0)
        if not last:
            jax.lax.fori_loop(0, ngroups // 2, m_desc, 0)
        span *= 2


def _sort_modulated(v, mask):
    b, h, s, dh = v.shape
    return pl.pallas_call(
        _bitonic_kernel,
        grid=(b, h),
        in_specs=[
            pl.BlockSpec((1, 1, s, dh), lambda i, j: (i, j, 0, 0)),
            pl.BlockSpec((8, dh), lambda i, j: (0, 0)),
        ],
        out_specs=pl.BlockSpec((1, 1, s, dh), lambda i, j: (i, j, 0, 0)),
        out_shape=jax.ShapeDtypeStruct(v.shape, v.dtype),
    )(v, mask)


@jax.jit
def kernel(q, k, v, col_descend):
    del q, k
    dh = v.shape[-1]
    cols = jnp.asarray(col_descend).reshape(-1).astype(jnp.int32)
    mask = jnp.zeros((8, dh), jnp.int32).at[0, cols].set(1)
    out = _sort_modulated(v, mask)
    return (out, out)
